# Initial kernel scaffold; baseline (speedup 1.0000x reference)
import jax
import jax.numpy as jnp
from jax.experimental import pallas as pl


def kernel(image_attn):
    raise NotImplementedError("write your pallas kernel here")


# speedup vs baseline: 1.1988x; 1.1988x over previous
import functools

import jax
import jax.numpy as jnp
from jax.experimental import pallas as pl
from jax.experimental.pallas import tpu as pltpu

_C_HI, _C_LO = 14, 4
_LANE = 128


def _reduce_body(x_ref, msum_ref, csum_ref, *, nblk, blocks_per_layer,
                 c_hi, c_lo):
    hb = pl.program_id(0)
    b = pl.program_id(1)

    @pl.when(b == 0)
    def _init():
        msum_ref[...] = jnp.zeros_like(msum_ref)
        csum_ref[...] = jnp.zeros_like(csum_ref)

    x = x_ref[...]
    r, vp = x.shape
    part = x.reshape(r // 8, 8, vp).astype(jnp.float32).sum(axis=0)
    msum_ref[0] += part

    layer = (hb * nblk + b) // blocks_per_layer
    sign = jnp.where(layer == c_hi, 1.0,
                     jnp.where(layer == c_lo, -1.0, 0.0)).astype(jnp.float32)

    @pl.when(sign != 0.0)
    def _contrast():
        csum_ref[0] += sign * part


def _attn_reduce(image_attn, c_hi, c_lo, block_rows=None):
    L, H, Q, Vp = image_attn.shape
    assert Vp % _LANE == 0
    assert L % 2 == 0, "megacore split over layer halves needs even L"

    rows_per_layer = H * Q
    rows = L * rows_per_layer
    rows_per_core = rows // 2

    if block_rows is None:
        block_rows = rows_per_layer
        while block_rows * Vp * 2 > 4 * 1024 * 1024:
            if block_rows % 2:
                break
            block_rows //= 2
    assert rows_per_layer % block_rows == 0
    nblk = rows_per_core // block_rows
    blocks_per_layer = rows_per_layer // block_rows

    flat = image_attn.reshape(rows, Vp)

    body = functools.partial(
        _reduce_body, nblk=nblk, blocks_per_layer=blocks_per_layer,
        c_hi=c_hi, c_lo=c_lo)

    msum, csum = pl.pallas_call(
        body,
        out_shape=(
            jax.ShapeDtypeStruct((2, 8, Vp), jnp.float32),
            jax.ShapeDtypeStruct((2, 8, Vp), jnp.float32),
        ),
        grid=(2, nblk),
        in_specs=[pl.BlockSpec((block_rows, Vp),
                               lambda hb, b: (hb * nblk + b, 0))],
        out_specs=(
            pl.BlockSpec((1, 8, Vp), lambda hb, b: (hb, 0, 0)),
            pl.BlockSpec((1, 8, Vp), lambda hb, b: (hb, 0, 0)),
        ),
        compiler_params=pltpu.CompilerParams(
            dimension_semantics=("parallel", "arbitrary")),
    )(flat)

    mean_attn = jnp.sum(msum, axis=(0, 1)) / float(rows)
    contr = jnp.sum(csum, axis=(0, 1)) / float(rows_per_layer)
    return mean_attn, jnp.maximum(contr, 0.0)


def kernel(image_attn):
    return _attn_reduce(image_attn, _C_HI, _C_LO)


# speedup vs baseline: 1.4721x; 1.2280x over previous
import functools

import jax
import jax.numpy as jnp
from jax.experimental import pallas as pl
from jax.experimental.pallas import tpu as pltpu

_C_HI, _C_LO = 14, 4
_LANE = 128


def _reduce_body(x_ref, msum_ref, csum_ref, *, nblk, blocks_per_layer,
                 c_hi, c_lo):
    hb = pl.program_id(0)
    b = pl.program_id(1)

    @pl.when(b == 0)
    def _init():
        msum_ref[...] = jnp.zeros_like(msum_ref)
        csum_ref[...] = jnp.zeros_like(csum_ref)

    x = x_ref[...]
    r, vp = x.shape
    part = x.reshape(r // 8, 8, vp).astype(jnp.float32).sum(axis=0)
    msum_ref[0] += part

    layer = (hb * nblk + b) // blocks_per_layer
    sign = jnp.where(layer == c_hi, 1.0,
                     jnp.where(layer == c_lo, -1.0, 0.0)).astype(jnp.float32)

    @pl.when(sign != 0.0)
    def _contrast():
        csum_ref[0] += sign * part


def _attn_reduce(image_attn, c_hi, c_lo, block_rows=None):
    L, H, Q, Vp = image_attn.shape
    assert Vp % _LANE == 0
    assert L % 2 == 0, "megacore split over layer halves needs even L"

    rows_per_layer = H * Q
    rows = L * rows_per_layer
    rows_per_core = rows // 2

    if block_rows is None:
        block_rows = rows_per_layer
        while block_rows * Vp * 2 > 4 * 1024 * 1024:
            if block_rows % 2:
                break
            block_rows //= 2
    assert rows_per_layer % block_rows == 0
    nblk = rows_per_core // block_rows
    blocks_per_layer = rows_per_layer // block_rows

    flat = image_attn.reshape(rows, Vp)

    body = functools.partial(
        _reduce_body, nblk=nblk, blocks_per_layer=blocks_per_layer,
        c_hi=c_hi, c_lo=c_lo)

    msum, csum = pl.pallas_call(
        body,
        out_shape=(
            jax.ShapeDtypeStruct((2, 8, Vp), jnp.float32),
            jax.ShapeDtypeStruct((2, 8, Vp), jnp.float32),
        ),
        grid=(2, nblk),
        in_specs=[pl.BlockSpec((block_rows, Vp),
                               lambda hb, b: (hb * nblk + b, 0))],
        out_specs=(
            pl.BlockSpec((1, 8, Vp), lambda hb, b: (hb, 0, 0)),
            pl.BlockSpec((1, 8, Vp), lambda hb, b: (hb, 0, 0)),
        ),
        compiler_params=pltpu.CompilerParams(
            dimension_semantics=("parallel", "arbitrary")),
    )(flat)

    mean_attn = jnp.sum(msum, axis=(0, 1)) / float(rows)
    contr = jnp.sum(csum, axis=(0, 1)) / float(rows_per_layer)
    return mean_attn, jnp.maximum(contr, 0.0)


def kernel(image_attn):
    return _attn_reduce(image_attn, _C_HI, _C_LO, block_rows=3584)
